# baseline (speedup 1.0000x reference)
import functools

import jax
import jax.numpy as jnp
from jax import lax
from jax.experimental import pallas as pl
from jax.experimental.pallas import tpu as pltpu
from jax.experimental.pallas import tpu_sc as plsc

_N = 50000
_C = 256
_G = 64
_EPS = 1e-05

_CHUNK = 2000
_NW = _N // _CHUNK
_SUB = 125
_NSUB = _CHUNK // _SUB
_R = 1000
_NB = _N // _R

_GP = 80
_TBL = _GP * 16


def _make_segstats():
    mesh = plsc.VectorSubcoreMesh(core_axis_name="c", subcore_axis_name="s")

    @functools.partial(
        pl.kernel,
        mesh=mesh,
        compiler_params=pltpu.CompilerParams(needs_layout_passes=False),
        out_type=jax.ShapeDtypeStruct((_NW, 3 * _GP), jnp.float32),
        scratch_types=[
            pltpu.VMEM((_CHUNK + 16,), jnp.int32),
            pltpu.VMEM((_SUB * _C,), jnp.float32),
            pltpu.VMEM((_SUB * _C,), jnp.float32),
            pltpu.VMEM((3 * _TBL,), jnp.float32),
            pltpu.VMEM((3 * _GP,), jnp.float32),
            pltpu.SemaphoreType.DMA,
            pltpu.SemaphoreType.DMA,
        ],
    )
    def segstats(x_hbm, b_hbm, out_hbm, bvec, xb0, xb1, acc, acc2, sem0, sem1):
        wid = lax.axis_index("s") * 2 + lax.axis_index("c")

        @pl.when(wid < _NW)
        def _():
            base = wid * _CHUNK
            pltpu.sync_copy(b_hbm.at[pl.ds(base, _CHUNK)], bvec.at[pl.ds(0, _CHUNK)])

            zeros = jnp.zeros((16,), jnp.float32)
            for j in range(3 * _TBL // 16):
                acc[pl.ds(j * 16, 16)] = zeros

            lane = lax.iota(jnp.int32, 16)
            ones = jnp.ones((16,), jnp.float32)
            for j in range(_CHUNK // 16):
                idx = bvec[pl.ds(j * 16, 16)] * 16 + lane
                plsc.addupdate_scatter(acc, [idx], ones)

            bufs = (xb0, xb1)
            sems = (sem0, sem1)

            def start(c):
                return pltpu.async_copy(
                    x_hbm.at[pl.ds((base + c * _SUB) * _C, _SUB * _C)],
                    bufs[c % 2], sems[c % 2])

            copies = {0: start(0)}
            for c in range(_NSUB):
                if c + 1 < _NSUB:
                    copies[c + 1] = start(c + 1)
                copies[c].wait()
                buf = bufs[c % 2]

                def row_body(r, _):
                    s = buf[pl.ds(r * _C, 16)]
                    q = s * s
                    for v in range(1, _C // 16):
                        t = buf[pl.ds(r * _C + v * 16, 16)]
                        s = s + t
                        q = q + t * t
                    b = bvec[pl.ds(c * _SUB + r, 16)][0]
                    bidx = b * 16 + lane
                    plsc.addupdate_scatter(acc, [bidx + _TBL], s)
                    plsc.addupdate_scatter(acc, [bidx + 2 * _TBL], q)
                    return _

                lax.fori_loop(0, _SUB, row_body, 0)

            for t0 in range(0, 3 * _GP, 16):
                base_idx = (t0 + lane) * 16
                v = plsc.load_gather(acc, [base_idx])
                for k in range(1, 16):
                    v = v + plsc.load_gather(acc, [base_idx + k])
                acc2[pl.ds(t0, 16)] = v

            pltpu.sync_copy(acc2, out_hbm.at[wid])

    return segstats


def _norm_kernel(x_ref, b_ref, p_ref, w_ref, bias_ref, o_ref, mi_tbl):
    i = pl.program_id(0)

    @pl.when(i == 0)
    def _():
        p = jnp.sum(p_ref[...], axis=0, keepdims=True)
        deg = p[:, 0:_G]
        s = p[:, _GP:_GP + _G]
        q = p[:, 2 * _GP:2 * _GP + _G]
        cnt = jnp.maximum(deg, 1.0) * _C
        mean = s / cnt
        var = jnp.maximum(q / cnt - mean * mean, 0.0)
        inv = 1.0 / (jnp.sqrt(var) + _EPS)
        mi_tbl[...] = jnp.concatenate([mean, inv], axis=0).T

    b = b_ref[0, 0, :]
    seg = jax.lax.broadcasted_iota(jnp.int32, (_R, _G), 1)
    oh = (seg == b[:, None]).astype(jnp.float32)
    mi = jnp.dot(oh, mi_tbl[...],
                 preferred_element_type=jnp.float32)
    xb = x_ref[...]
    o_ref[...] = ((xb - mi[:, 0:1]) * mi[:, 1:2]) * w_ref[...] + bias_ref[...]


def kernel(x, batch, weight, bias):
    batch = batch.astype(jnp.int32)
    batch3 = batch.reshape(_NB, 1, _R)
    xflat = x.reshape(_N * _C)

    partials = _make_segstats()(xflat, batch)

    out = pl.pallas_call(
        _norm_kernel,
        grid=(_NB,),
        in_specs=[
            pl.BlockSpec((_R, _C), lambda i: (i, 0)),
            pl.BlockSpec((1, 1, _R), lambda i: (i, 0, 0)),
            pl.BlockSpec((_NW, 3 * _GP), lambda i: (0, 0)),
            pl.BlockSpec((1, _C), lambda i: (0, 0)),
            pl.BlockSpec((1, _C), lambda i: (0, 0)),
        ],
        out_specs=pl.BlockSpec((_R, _C), lambda i: (i, 0)),
        out_shape=jax.ShapeDtypeStruct((_N, _C), jnp.float32),
        scratch_shapes=[pltpu.VMEM((_G, 2), jnp.float32)],
    )(x, batch3, partials, weight, bias)
    return out


# speedup vs baseline: 1.8676x; 1.8676x over previous
import functools

import jax
import jax.numpy as jnp
from jax import lax
from jax.experimental import pallas as pl
from jax.experimental.pallas import tpu as pltpu
from jax.experimental.pallas import tpu_sc as plsc

_N = 50000
_C = 256
_G = 64
_EPS = 1e-05

_CHUNK = 2000
_NW = _N // _CHUNK
_R = 2000
_NB = _N // _R

_GP = 80
_TBL = _GP * 16


def _rowstats_kernel(x_ref, o_ref):
    xb = x_ref[...]
    ones_c = jnp.ones((_C, 1), jnp.float32)
    dn = (((0,), (1,)), ((), ()))
    rs = lax.dot_general(ones_c, xb, dn,
                         preferred_element_type=jnp.float32)
    rq = lax.dot_general(ones_c, xb * xb, dn,
                         preferred_element_type=jnp.float32)
    o_ref[...] = jnp.concatenate([rs, rq], axis=0).reshape(1, 2, _CHUNK)


def _make_segsum():
    mesh = plsc.VectorSubcoreMesh(core_axis_name="c", subcore_axis_name="s")

    @functools.partial(
        pl.kernel,
        mesh=mesh,
        compiler_params=pltpu.CompilerParams(needs_layout_passes=False),
        out_type=jax.ShapeDtypeStruct((_NW, 3 * _GP), jnp.float32),
        scratch_types=[
            pltpu.VMEM((_CHUNK,), jnp.int32),
            pltpu.VMEM((_CHUNK,), jnp.float32),
            pltpu.VMEM((_CHUNK,), jnp.float32),
            pltpu.VMEM((3 * _TBL,), jnp.float32),
            pltpu.VMEM((3 * _GP,), jnp.float32),
        ],
    )
    def segsum(rsq_hbm, b_hbm, out_hbm, bvec, rsv, rqv, acc, acc2):
        wid = lax.axis_index("s") * 2 + lax.axis_index("c")

        @pl.when(wid < _NW)
        def _():
            pltpu.sync_copy(b_hbm.at[pl.ds(wid * _CHUNK, _CHUNK)], bvec)
            pltpu.sync_copy(rsq_hbm.at[wid, 0], rsv)
            pltpu.sync_copy(rsq_hbm.at[wid, 1], rqv)

            zeros = jnp.zeros((16,), jnp.float32)
            for j in range(3 * _TBL // 16):
                acc[pl.ds(j * 16, 16)] = zeros

            lane = lax.iota(jnp.int32, 16)
            ones = jnp.ones((16,), jnp.float32)
            for j in range(_CHUNK // 16):
                sl = pl.ds(j * 16, 16)
                idx = bvec[sl] * 16 + lane
                plsc.addupdate_scatter(acc, [idx], ones)
                plsc.addupdate_scatter(acc, [idx + _TBL], rsv[sl])
                plsc.addupdate_scatter(acc, [idx + 2 * _TBL], rqv[sl])

            for t0 in range(0, 3 * _GP, 16):
                base_idx = (t0 + lane) * 16
                v = plsc.load_gather(acc, [base_idx])
                for k in range(1, 16):
                    v = v + plsc.load_gather(acc, [base_idx + k])
                acc2[pl.ds(t0, 16)] = v

            pltpu.sync_copy(acc2, out_hbm.at[wid])

    return segsum


def _norm_kernel(x_ref, b_ref, p_ref, w_ref, bias_ref, o_ref, mi_tbl):
    i = pl.program_id(0)

    @pl.when(i == 0)
    def _():
        p = jnp.sum(p_ref[...], axis=0, keepdims=True)
        deg = p[:, 0:_G]
        s = p[:, _GP:_GP + _G]
        q = p[:, 2 * _GP:2 * _GP + _G]
        cnt = jnp.maximum(deg, 1.0) * _C
        mean = s / cnt
        var = jnp.maximum(q / cnt - mean * mean, 0.0)
        inv = 1.0 / (jnp.sqrt(var) + _EPS)
        mi_tbl[...] = jnp.concatenate([mean, inv], axis=0).T

    b = b_ref[0, 0, :]
    seg = jax.lax.broadcasted_iota(jnp.int32, (_R, _G), 1)
    oh = (seg == b[:, None]).astype(jnp.float32)
    mi = jnp.dot(oh, mi_tbl[...],
                 preferred_element_type=jnp.float32)
    xb = x_ref[...]
    o_ref[...] = ((xb - mi[:, 0:1]) * mi[:, 1:2]) * w_ref[...] + bias_ref[...]


def kernel(x, batch, weight, bias):
    batch = batch.astype(jnp.int32)
    batch3 = batch.reshape(_NB, 1, _R)

    rsq = pl.pallas_call(
        _rowstats_kernel,
        grid=(_NW,),
        in_specs=[pl.BlockSpec((_CHUNK, _C), lambda i: (i, 0))],
        out_specs=pl.BlockSpec((1, 2, _CHUNK), lambda i: (i, 0, 0)),
        out_shape=jax.ShapeDtypeStruct((_NW, 2, _CHUNK), jnp.float32),
    )(x)

    partials = _make_segsum()(rsq, batch)

    out = pl.pallas_call(
        _norm_kernel,
        grid=(_NB,),
        in_specs=[
            pl.BlockSpec((_R, _C), lambda i: (i, 0)),
            pl.BlockSpec((1, 1, _R), lambda i: (i, 0, 0)),
            pl.BlockSpec((_NW, 3 * _GP), lambda i: (0, 0)),
            pl.BlockSpec((1, _C), lambda i: (0, 0)),
            pl.BlockSpec((1, _C), lambda i: (0, 0)),
        ],
        out_specs=pl.BlockSpec((_R, _C), lambda i: (i, 0)),
        out_shape=jax.ShapeDtypeStruct((_N, _C), jnp.float32),
        scratch_shapes=[pltpu.VMEM((_G, 2), jnp.float32)],
    )(x, batch3, partials, weight, bias)
    return out
